# baseline (speedup 1.0000x reference)
import jax
import jax.numpy as jnp
from jax.experimental import pallas as pl
from jax.experimental.pallas import tpu as pltpu

PERCENT_ON = 0.1


def _make_body(BM, N, NT, G, K):
    TN = N // NT

    def _body(x_ref, w_ref, b_ref, o_ref, ha, hb):
        i = pl.program_id(0)

        def work(cur, prv):
            z = prv[...]
            rmin = jnp.min(z, axis=1, keepdims=True)
            rmax = jnp.max(z, axis=1, keepdims=True)
            inv = 1.0 / (rmax - rmin)
            hn = (z - rmin) * inv
            u = jax.lax.bitcast_convert_type(hn, jnp.int32)

            def iter_body(j, t):
                wj = w_ref[pl.ds(j * TN, TN), :]
                hj = jax.lax.dot_general(
                    x_ref[...], wj, (((1,), (1,)), ((), ())),
                    preferred_element_type=jnp.float32)
                cur[:, pl.ds(j * TN, TN)] = hj + b_ref[:, pl.ds(j * TN, TN)]
                bit = jnp.maximum(29 - j, 0)
                cand = t | (jnp.int32(1) << bit)
                cnt = jnp.sum((u >= cand).astype(jnp.int32), axis=1,
                              keepdims=True)
                return jnp.where(cnt >= K, cand, t)

            t = jax.lax.fori_loop(0, NT, iter_body,
                                  jnp.zeros((BM, 1), jnp.int32))

            f = jnp.where(u >= t, hn, 0.0)
            ssq = jnp.sum(f * f, axis=1, keepdims=True)
            o_ref[...] = f / jnp.maximum(jnp.sqrt(ssq), 1e-12)

        @pl.when(jax.lax.rem(i, 2) == 0)
        def _even():
            work(ha, hb)

        @pl.when(jax.lax.rem(i, 2) == 1)
        def _odd():
            work(hb, ha)

    return _body


def kernel(x, W, b):
    B, D = x.shape
    N = W.shape[0]
    K = int(round(N * PERCENT_ON))
    BM = min(128, B)
    G = B // BM
    NT = 32
    grid = (G + 1,)
    gl = G - 1
    return pl.pallas_call(
        _make_body(BM, N, NT, G, K),
        grid=grid,
        in_specs=[
            pl.BlockSpec((BM, D), lambda i: (jnp.minimum(i, gl), 0)),
            pl.BlockSpec((N, D), lambda i: (0, 0)),
            pl.BlockSpec((1, N), lambda i: (0, 0)),
        ],
        out_specs=pl.BlockSpec((BM, N), lambda i: (jnp.maximum(i - 1, 0), 0)),
        out_shape=jax.ShapeDtypeStruct((B, N), jnp.float32),
        scratch_shapes=[
            pltpu.VMEM((BM, N), jnp.float32),
            pltpu.VMEM((BM, N), jnp.float32),
        ],
        compiler_params=pltpu.CompilerParams(
            dimension_semantics=("arbitrary",),
        ),
    )(x, W, b.reshape(1, N))
